# Initial kernel scaffold; baseline (speedup 1.0000x reference)
import functools

import jax
import jax.numpy as jnp
from jax import lax
from jax.experimental import pallas as pl
from jax.experimental.pallas import tpu as pltpu
from jax.experimental.pallas import tpu_sc as plsc

LAM = 1.0
N_ITERS = 10
A = 4
D = 8
B = 2
BD = B * D

NC = 2
NS = 16
W = NC * NS
BLK = 256
HB = 128
NH = BLK // HB


def _build_matvec(n_nodes: int, e_pad: int, chunks: int):
    acc_pt = -(-n_nodes // NS)
    acc_pt = -(-acc_pt // BLK) * BLK
    acc_rows = acc_pt * NS

    mesh = plsc.VectorSubcoreMesh(core_axis_name="c", subcore_axis_name="s")

    @functools.partial(
        pl.kernel,
        out_type=jax.ShapeDtypeStruct((NC, acc_rows, BD), jnp.float32),
        mesh=mesh,
        scratch_types=[
            pltpu.VMEM((BLK, A * D), jnp.float32),
            pltpu.VMEM((BLK, A * D), jnp.float32),
            pltpu.VMEM((NH, HB), jnp.int32),
            pltpu.VMEM((NH, HB), jnp.int32),
            pltpu.VMEM((BLK, BD), jnp.float32),
            pltpu.VMEM((BLK, BD), jnp.float32),
            pltpu.VMEM((BLK, BD), jnp.float32),
            pltpu.VMEM((BLK, BD), jnp.float32),
            pltpu.VMEM_SHARED((acc_rows, BD), jnp.float32),
        ],
    )
    def matvec_sc(p_h, src_h, dst_h, rs_h, rd_h, out_h,
                  rsv, rdv, sidx, didx, psv, pdv, csv, cdv, acc):
        cid = lax.axis_index("c")
        sid = lax.axis_index("s")
        w = cid * NS + sid

        zvec = jnp.zeros((BD,), jnp.float32)

        @pl.loop(0, BLK)
        def _zero_buf(i):
            csv[i] = zvec

        z0 = sid * acc_pt

        @pl.loop(0, acc_pt // BLK)
        def _zero_acc(k):
            pltpu.sync_copy(csv, acc.at[pl.ds(z0 + k * BLK, BLK)])

        plsc.subcore_barrier()

        e16 = jnp.arange(16, dtype=jnp.int32)

        @pl.loop(0, chunks)
        def _chunk(c):
            g = w * chunks + c
            base = g * BLK
            row0 = g * NH
            pltpu.sync_copy(src_h.at[pl.ds(row0, NH)], sidx)
            pltpu.sync_copy(dst_h.at[pl.ds(row0, NH)], didx)
            pltpu.sync_copy(rs_h.at[pl.ds(base, BLK)], rsv)
            pltpu.sync_copy(rd_h.at[pl.ds(base, BLK)], rdv)
            for h in range(NH):
                pltpu.sync_copy(p_h.at[sidx.at[h]],
                                psv.at[pl.ds(h * HB, HB)])
                pltpu.sync_copy(p_h.at[didx.at[h]],
                                pdv.at[pl.ds(h * HB, HB)])

            @pl.loop(0, BLK // 16)
            def _blk(j):
                rows = j * 16 + e16

                def col(t):
                    return jnp.full((16,), t, jnp.int32)

                psT = [plsc.load_gather(psv, [rows, col(t)])
                       for t in range(BD)]
                pdT = [plsc.load_gather(pdv, [rows, col(t)])
                       for t in range(BD)]

                r = [[None] * A for _ in range(B)]
                for a in range(A):
                    for d in range(D):
                        k = a * D + d
                        rsk = plsc.load_gather(rsv, [rows, col(k)])
                        rdk = plsc.load_gather(rdv, [rows, col(k)])
                        for b in range(B):
                            t = rsk * psT[b * D + d] - rdk * pdT[b * D + d]
                            if r[b][a] is None:
                                r[b][a] = t
                            else:
                                r[b][a] = r[b][a] + t

                for d in range(D):
                    rs_cols = [plsc.load_gather(rsv, [rows, col(a * D + d)])
                               for a in range(A)]
                    rd_cols = [plsc.load_gather(rdv, [rows, col(a * D + d)])
                               for a in range(A)]
                    for b in range(B):
                        cs = rs_cols[0] * r[b][0]
                        cd = rd_cols[0] * r[b][0]
                        for a in range(1, A):
                            cs = cs + rs_cols[a] * r[b][a]
                            cd = cd + rd_cols[a] * r[b][a]
                        plsc.store_scatter(csv, [rows, col(b * D + d)], cs)
                        plsc.store_scatter(cdv, [rows, col(b * D + d)], -cd)

            for h in range(NH):
                pltpu.sync_copy(csv.at[pl.ds(h * HB, HB)],
                                acc.at[sidx.at[h]], add=True)
                pltpu.sync_copy(cdv.at[pl.ds(h * HB, HB)],
                                acc.at[didx.at[h]], add=True)

        plsc.subcore_barrier()
        pltpu.sync_copy(acc.at[pl.ds(z0, acc_pt)],
                        out_h.at[cid].at[pl.ds(z0, acc_pt)])

    return matvec_sc, acc_rows


def kernel(src, dst, R_src, R_dst, c0):
    e = src.shape[0]
    n = c0.shape[1]
    chunks = -(-e // (W * BLK))
    e_pad = W * chunks * BLK

    matvec_sc, acc_rows = _build_matvec(n, e_pad, chunks)

    pad = e_pad - e
    src_p = jnp.pad(src.astype(jnp.int32), (0, pad)).reshape(e_pad // HB, HB)
    dst_p = jnp.pad(dst.astype(jnp.int32), (0, pad)).reshape(e_pad // HB, HB)
    rs_p = jnp.pad(R_src.reshape(e, A * D), ((0, pad), (0, 0)))
    rd_p = jnp.pad(R_dst.reshape(e, A * D), ((0, pad), (0, 0)))

    def matvec(p):
        out = matvec_sc(p, src_p, dst_p, rs_p, rd_p)
        return p + LAM * (out[0, :n] + out[1, :n])

    def bdot(u, v):
        s = (u * v).reshape(n, B, D).sum(axis=(0, 2))
        return jnp.repeat(s, D)[None, :]

    b = c0.astype(jnp.float32).transpose(1, 0, 2).reshape(n, BD)
    x = b
    r = b - matvec(x)
    p = r
    rsold = bdot(r, r)
    for _ in range(N_ITERS):
        ap = matvec(p)
        denom = bdot(p, ap) + 1e-12
        alpha = rsold / denom
        x = x + alpha * p
        r = r - alpha * ap
        rsnew = bdot(r, r)
        p = r + (rsnew / (rsold + 1e-12)) * p
        rsold = rsnew
    return x.reshape(n, B, D).transpose(1, 0, 2)


# speedup vs baseline: 64.0441x; 64.0441x over previous
import functools

import jax
import jax.numpy as jnp
from jax import lax
from jax.experimental import pallas as pl
from jax.experimental.pallas import tpu as pltpu
from jax.experimental.pallas import tpu_sc as plsc

LAM = 1.0
N_ITERS = 10
A = 4
D = 8
B = 2
BD = B * D

NC = 2
NS = 16
W = NC * NS
BLK = 256
HB = 128
NH = BLK // HB


def _build_matvec(n_nodes: int, e_pad: int, chunks: int):
    acc_pt = -(-n_nodes // NS)
    acc_pt = -(-acc_pt // BLK) * BLK
    acc_rows = acc_pt * NS

    mesh = plsc.VectorSubcoreMesh(core_axis_name="c", subcore_axis_name="s",
                                  num_cores=NC, num_subcores=NS)

    def _round_bf16(v):
        u = plsc.bitcast(v, jnp.uint32)
        u = (u + ((u >> 16) & jnp.uint32(1)) + jnp.uint32(0x7FFF))
        u = u & jnp.uint32(0xFFFF0000)
        return plsc.bitcast(u, jnp.float32)

    @functools.partial(
        pl.kernel,
        out_type=jax.ShapeDtypeStruct((NC, acc_rows, BD), jnp.float32),
        mesh=mesh,
        compiler_params=pltpu.CompilerParams(needs_layout_passes=False,
                                             use_tc_tiling_on_sc=False),
        scratch_types=[
            pltpu.VMEM((BLK, A * D), jnp.float32),
            pltpu.VMEM((BLK, A * D), jnp.float32),
            pltpu.VMEM((NH, HB), jnp.int32),
            pltpu.VMEM((NH, HB), jnp.int32),
            pltpu.VMEM((BLK, BD), jnp.float32),
            pltpu.VMEM((BLK, BD), jnp.float32),
            pltpu.VMEM((BLK, BD), jnp.float32),
            pltpu.VMEM((BLK, BD), jnp.float32),
            pltpu.VMEM_SHARED((acc_rows, BD), jnp.float32),
        ],
    )
    def matvec_sc(p_h, src_h, dst_h, rs_h, rd_h, out_h,
                  rsv, rdv, sidx, didx, psv, pdv, csv, cdv, acc):
        cid = lax.axis_index("c")
        sid = lax.axis_index("s")
        w = cid * NS + sid

        zvec = jnp.zeros((BD,), jnp.float32)

        @pl.loop(0, BLK)
        def _zero_buf(i):
            csv[i] = zvec

        z0 = sid * acc_pt

        @pl.loop(0, acc_pt // BLK)
        def _zero_acc(k):
            pltpu.sync_copy(csv, acc.at[pl.ds(z0 + k * BLK, BLK)])

        plsc.subcore_barrier()

        e16 = jnp.arange(16, dtype=jnp.int32)

        @pl.loop(0, chunks)
        def _chunk(c):
            g = w * chunks + c
            base = g * BLK
            row0 = g * NH
            pltpu.sync_copy(src_h.at[pl.ds(row0, NH)], sidx)
            pltpu.sync_copy(dst_h.at[pl.ds(row0, NH)], didx)
            pltpu.sync_copy(rs_h.at[pl.ds(base, BLK)], rsv)
            pltpu.sync_copy(rd_h.at[pl.ds(base, BLK)], rdv)
            for h in range(NH):
                pltpu.sync_copy(p_h.at[sidx.at[h]],
                                psv.at[pl.ds(h * HB, HB)])
                pltpu.sync_copy(p_h.at[didx.at[h]],
                                pdv.at[pl.ds(h * HB, HB)])

            @pl.loop(0, BLK // 16)
            def _blk(j):
                rows = j * 16 + e16

                def col(t):
                    return jnp.full((16,), t, jnp.int32)

                psT = [plsc.load_gather(psv, [rows, col(t)])
                       for t in range(BD)]
                pdT = [plsc.load_gather(pdv, [rows, col(t)])
                       for t in range(BD)]

                r = [[None] * A for _ in range(B)]
                for a in range(A):
                    for d in range(D):
                        k = a * D + d
                        rsk = plsc.load_gather(rsv, [rows, col(k)])
                        rdk = plsc.load_gather(rdv, [rows, col(k)])
                        for b in range(B):
                            t = rsk * psT[b * D + d] - rdk * pdT[b * D + d]
                            if r[b][a] is None:
                                r[b][a] = t
                            else:
                                r[b][a] = r[b][a] + t

                for b in range(B):
                    for a in range(A):
                        r[b][a] = _round_bf16(r[b][a])

                for d in range(D):
                    rs_cols = [plsc.load_gather(rsv, [rows, col(a * D + d)])
                               for a in range(A)]
                    rd_cols = [plsc.load_gather(rdv, [rows, col(a * D + d)])
                               for a in range(A)]
                    for b in range(B):
                        cs = rs_cols[0] * r[b][0]
                        cd = rd_cols[0] * r[b][0]
                        for a in range(1, A):
                            cs = cs + rs_cols[a] * r[b][a]
                            cd = cd + rd_cols[a] * r[b][a]
                        plsc.store_scatter(csv, [rows, col(b * D + d)], cs)
                        plsc.store_scatter(cdv, [rows, col(b * D + d)], -cd)

            for h in range(NH):
                pltpu.sync_copy(csv.at[pl.ds(h * HB, HB)],
                                acc.at[sidx.at[h]], add=True)
                pltpu.sync_copy(cdv.at[pl.ds(h * HB, HB)],
                                acc.at[didx.at[h]], add=True)

        plsc.subcore_barrier()
        pltpu.sync_copy(acc.at[pl.ds(z0, acc_pt)],
                        out_h.at[cid].at[pl.ds(z0, acc_pt)])

    return matvec_sc, acc_rows


def kernel(src, dst, R_src, R_dst, c0):
    e = src.shape[0]
    n = c0.shape[1]
    chunks = -(-e // (W * BLK))
    e_pad = W * chunks * BLK

    matvec_sc, acc_rows = _build_matvec(n, e_pad, chunks)

    def bf(v):
        u = jax.lax.bitcast_convert_type(v, jnp.uint32)
        u = (u + ((u >> 16) & jnp.uint32(1)) + jnp.uint32(0x7FFF))
        u = u & jnp.uint32(0xFFFF0000)
        return jax.lax.bitcast_convert_type(u, jnp.float32)

    pad = e_pad - e
    src_p = jnp.pad(src.astype(jnp.int32), (0, pad)).reshape(e_pad // HB, HB)
    dst_p = jnp.pad(dst.astype(jnp.int32), (0, pad)).reshape(e_pad // HB, HB)
    rs_p = jnp.pad(bf(R_src).reshape(e, A * D), ((0, pad), (0, 0)))
    rd_p = jnp.pad(bf(R_dst).reshape(e, A * D), ((0, pad), (0, 0)))

    def matvec(p):
        out = matvec_sc(bf(p), src_p, dst_p, rs_p, rd_p)
        return p + LAM * (out[0, :n] + out[1, :n])

    def bdot(u, v):
        s = (u * v).reshape(n, B, D).sum(axis=(0, 2))
        return jnp.repeat(s, D)[None, :]

    b = c0.astype(jnp.float32).transpose(1, 0, 2).reshape(n, BD)
    x = b
    r = b - matvec(x)
    p = r
    rsold = bdot(r, r)
    for _ in range(N_ITERS):
        ap = matvec(p)
        denom = bdot(p, ap) + 1e-12
        alpha = rsold / denom
        x = x + alpha * p
        r = r - alpha * ap
        rsnew = bdot(r, r)
        p = r + (rsnew / (rsold + 1e-12)) * p
        rsold = rsnew
    return x.reshape(n, B, D).transpose(1, 0, 2)


# speedup vs baseline: 82.6232x; 1.2901x over previous
import functools

import jax
import jax.numpy as jnp
from jax import lax
from jax.experimental import pallas as pl
from jax.experimental.pallas import tpu as pltpu
from jax.experimental.pallas import tpu_sc as plsc

LAM = 1.0
N_ITERS = 10
A = 4
D = 8
B = 2
BD = B * D

NC = 2
NS = 16
W = NC * NS
BLK = 128


def _build_matvec(n_nodes: int, e_pad: int, chunks: int):
    acc_pt = -(-n_nodes // NS)
    acc_pt = -(-acc_pt // BLK) * BLK
    acc_rows = acc_pt * NS
    half = chunks // 2

    mesh = plsc.VectorSubcoreMesh(core_axis_name="c", subcore_axis_name="s",
                                  num_cores=NC, num_subcores=NS)

    @functools.partial(
        pl.kernel,
        out_type=jax.ShapeDtypeStruct((NC, acc_rows, BD), jnp.float32),
        mesh=mesh,
        compiler_params=pltpu.CompilerParams(needs_layout_passes=False,
                                             use_tc_tiling_on_sc=False),
        scratch_types=[
            pltpu.VMEM((BLK, A * D), jnp.float32),
            pltpu.VMEM((BLK, A * D), jnp.float32),
            pltpu.VMEM((BLK, A * D), jnp.float32),
            pltpu.VMEM((BLK, A * D), jnp.float32),
            pltpu.VMEM((BLK, BD), jnp.float32),
            pltpu.VMEM((BLK, BD), jnp.float32),
            pltpu.VMEM((BLK, BD), jnp.float32),
            pltpu.VMEM((BLK, BD), jnp.float32),
            pltpu.VMEM((BLK, BD), jnp.float32),
            pltpu.VMEM((BLK, BD), jnp.float32),
            pltpu.VMEM((BLK, BD), jnp.float32),
            pltpu.VMEM((BLK, BD), jnp.float32),
            pltpu.VMEM((half, BLK), jnp.int32),
            pltpu.VMEM((half, BLK), jnp.int32),
            pltpu.VMEM_SHARED((acc_rows, BD), jnp.float32),
            pltpu.SemaphoreType.DMA,
            pltpu.SemaphoreType.DMA,
            pltpu.SemaphoreType.DMA,
            pltpu.SemaphoreType.DMA,
        ],
    )
    def matvec_sc(p_h, src_h, dst_h, rs_h, rd_h, out_h,
                  rsv0, rsv1, rdv0, rdv1, psv0, psv1, pdv0, pdv1,
                  csv0, csv1, cdv0, cdv1, sidx, didx, acc,
                  sem_in0, sem_in1, sem_out0, sem_out1):
        cid = lax.axis_index("c")
        sid = lax.axis_index("s")
        w = cid * NS + sid

        rsv = (rsv0, rsv1)
        rdv = (rdv0, rdv1)
        psv = (psv0, psv1)
        pdv = (pdv0, pdv1)
        csv = (csv0, csv1)
        cdv = (cdv0, cdv1)
        sem_in = (sem_in0, sem_in1)
        sem_out = (sem_out0, sem_out1)

        def _round_bf16(v):
            u = plsc.bitcast(v, jnp.uint32)
            u = (u + ((u >> 16) & jnp.uint32(1)) + jnp.uint32(0x7FFF))
            u = u & jnp.uint32(0xFFFF0000)
            return plsc.bitcast(u, jnp.float32)

        zvec = jnp.zeros((BD,), jnp.float32)

        @pl.loop(0, BLK)
        def _zero_buf(i):
            csv0[i] = zvec

        z0 = sid * acc_pt

        @pl.loop(0, acc_pt // BLK)
        def _zero_acc(k):
            pltpu.sync_copy(csv0, acc.at[pl.ds(z0 + k * BLK, BLK)])

        def issue_in(g, c, s):
            base = (w * chunks + g) * BLK
            pltpu.async_copy(rs_h.at[pl.ds(base, BLK)], rsv[s], sem_in[s])
            pltpu.async_copy(rd_h.at[pl.ds(base, BLK)], rdv[s], sem_in[s])
            pltpu.async_copy(p_h.at[sidx.at[c]], psv[s], sem_in[s])
            pltpu.async_copy(p_h.at[didx.at[c]], pdv[s], sem_in[s])

        def wait_in(s):
            pltpu.make_async_copy(rs_h.at[pl.ds(0, BLK)], rsv[s],
                                  sem_in[s]).wait()
            pltpu.make_async_copy(rd_h.at[pl.ds(0, BLK)], rdv[s],
                                  sem_in[s]).wait()
            pltpu.make_async_copy(p_h.at[pl.ds(0, BLK)], psv[s],
                                  sem_in[s]).wait()
            pltpu.make_async_copy(p_h.at[pl.ds(0, BLK)], pdv[s],
                                  sem_in[s]).wait()

        def issue_out(c, s):
            pltpu.async_copy(csv[s], acc.at[sidx.at[c]], sem_out[s],
                             add=True)
            pltpu.async_copy(cdv[s], acc.at[didx.at[c]], sem_out[s],
                             add=True)

        def wait_out(s):
            pltpu.make_async_copy(csv[s], acc.at[pl.ds(0, BLK)],
                                  sem_out[s]).wait()
            pltpu.make_async_copy(cdv[s], acc.at[pl.ds(0, BLK)],
                                  sem_out[s]).wait()

        e16 = jnp.arange(16, dtype=jnp.int32)

        def compute_chunk(s):
            @pl.loop(0, BLK // 16)
            def _blk(j):
                rows = j * 16 + e16

                def col(t):
                    return jnp.full((16,), t, jnp.int32)

                psT = [plsc.load_gather(psv[s], [rows, col(t)])
                       for t in range(BD)]
                pdT = [plsc.load_gather(pdv[s], [rows, col(t)])
                       for t in range(BD)]

                r = [[None] * A for _ in range(B)]
                for a in range(A):
                    for d in range(D):
                        k = a * D + d
                        rsk = plsc.load_gather(rsv[s], [rows, col(k)])
                        rdk = plsc.load_gather(rdv[s], [rows, col(k)])
                        for b in range(B):
                            t = rsk * psT[b * D + d] - rdk * pdT[b * D + d]
                            if r[b][a] is None:
                                r[b][a] = t
                            else:
                                r[b][a] = r[b][a] + t

                for b in range(B):
                    for a in range(A):
                        r[b][a] = _round_bf16(r[b][a])

                for d in range(D):
                    rs_cols = [plsc.load_gather(rsv[s], [rows, col(a * D + d)])
                               for a in range(A)]
                    rd_cols = [plsc.load_gather(rdv[s], [rows, col(a * D + d)])
                               for a in range(A)]
                    for b in range(B):
                        cs = rs_cols[0] * r[b][0]
                        cd = rd_cols[0] * r[b][0]
                        for a in range(1, A):
                            cs = cs + rs_cols[a] * r[b][a]
                            cd = cd + rd_cols[a] * r[b][a]
                        plsc.store_scatter(csv[s], [rows, col(b * D + d)], cs)
                        plsc.store_scatter(cdv[s], [rows, col(b * D + d)], -cd)

        plsc.subcore_barrier()

        def run_pass(g0):
            pltpu.sync_copy(src_h.at[pl.ds(w * chunks + g0, half)], sidx)
            pltpu.sync_copy(dst_h.at[pl.ds(w * chunks + g0, half)], didx)
            issue_in(g0, 0, 0)

            @pl.loop(0, half // 2)
            def _pair(i):
                c0 = i * 2
                wait_in(0)
                issue_in(g0 + c0 + 1, c0 + 1, 1)

                @pl.when(i > 0)
                def _():
                    wait_out(0)

                compute_chunk(0)
                issue_out(c0, 0)

                wait_in(1)

                @pl.when(c0 + 2 < half)
                def _():
                    issue_in(g0 + c0 + 2, c0 + 2, 0)

                @pl.when(i > 0)
                def _():
                    wait_out(1)

                compute_chunk(1)
                issue_out(c0 + 1, 1)

            wait_out(0)
            wait_out(1)

        run_pass(0)
        run_pass(half)

        plsc.subcore_barrier()
        pltpu.sync_copy(acc.at[pl.ds(z0, acc_pt)],
                        out_h.at[cid].at[pl.ds(z0, acc_pt)])

    return matvec_sc, acc_rows


def kernel(src, dst, R_src, R_dst, c0):
    e = src.shape[0]
    n = c0.shape[1]
    chunks = 4 * (-(-e // (W * BLK * 4)))
    e_pad = W * chunks * BLK

    matvec_sc, acc_rows = _build_matvec(n, e_pad, chunks)

    def bf(v):
        u = jax.lax.bitcast_convert_type(v, jnp.uint32)
        u = (u + ((u >> 16) & jnp.uint32(1)) + jnp.uint32(0x7FFF))
        u = u & jnp.uint32(0xFFFF0000)
        return jax.lax.bitcast_convert_type(u, jnp.float32)

    pad = e_pad - e
    src_p = jnp.pad(src.astype(jnp.int32), (0, pad)).reshape(e_pad // BLK, BLK)
    dst_p = jnp.pad(dst.astype(jnp.int32), (0, pad)).reshape(e_pad // BLK, BLK)
    rs_p = jnp.pad(bf(R_src).reshape(e, A * D), ((0, pad), (0, 0)))
    rd_p = jnp.pad(bf(R_dst).reshape(e, A * D), ((0, pad), (0, 0)))

    def matvec(p):
        out = matvec_sc(bf(p), src_p, dst_p, rs_p, rd_p)
        return p + LAM * (out[0, :n] + out[1, :n])

    def bdot(u, v):
        s = (u * v).reshape(n, B, D).sum(axis=(0, 2))
        return jnp.repeat(s, D)[None, :]

    b = c0.astype(jnp.float32).transpose(1, 0, 2).reshape(n, BD)
    x = b
    r = b - matvec(x)
    p = r
    rsold = bdot(r, r)
    for _ in range(N_ITERS):
        ap = matvec(p)
        denom = bdot(p, ap) + 1e-12
        alpha = rsold / denom
        x = x + alpha * p
        r = r - alpha * ap
        rsnew = bdot(r, r)
        p = r + (rsnew / (rsold + 1e-12)) * p
        rsold = rsnew
    return x.reshape(n, B, D).transpose(1, 0, 2)


# speedup vs baseline: 242.1807x; 2.9311x over previous
import functools

import jax
import jax.numpy as jnp
from jax import lax
from jax.experimental import pallas as pl
from jax.experimental.pallas import tpu as pltpu
from jax.experimental.pallas import tpu_sc as plsc

LAM = 1.0
N_ITERS = 10
A = 4
D = 8
B = 2
BD = B * D

NC = 2
NS = 16
W = NC * NS
BLK = 128


def _build_matvec(n_nodes: int, e_pad: int, chunks: int):
    acc_pt = -(-n_nodes // NS)
    acc_pt = -(-acc_pt // BLK) * BLK
    acc_rows = acc_pt * NS
    half = chunks // 2

    mesh = plsc.VectorSubcoreMesh(core_axis_name="c", subcore_axis_name="s",
                                  num_cores=NC, num_subcores=NS)

    @functools.partial(
        pl.kernel,
        out_type=jax.ShapeDtypeStruct((NC, acc_rows, BD), jnp.float32),
        mesh=mesh,
        compiler_params=pltpu.CompilerParams(needs_layout_passes=False,
                                             use_tc_tiling_on_sc=False),
        scratch_types=[
            pltpu.VMEM((A * D, BLK), jnp.float32),
            pltpu.VMEM((A * D, BLK), jnp.float32),
            pltpu.VMEM((A * D, BLK), jnp.float32),
            pltpu.VMEM((A * D, BLK), jnp.float32),
            pltpu.VMEM((BLK, BD), jnp.float32),
            pltpu.VMEM((BLK, BD), jnp.float32),
            pltpu.VMEM((BLK, BD), jnp.float32),
            pltpu.VMEM((BLK, BD), jnp.float32),
            pltpu.VMEM((BLK, BD), jnp.float32),
            pltpu.VMEM((BLK, BD), jnp.float32),
            pltpu.VMEM((BLK, BD), jnp.float32),
            pltpu.VMEM((BLK, BD), jnp.float32),
            pltpu.VMEM((half, BLK), jnp.int32),
            pltpu.VMEM((half, BLK), jnp.int32),
            pltpu.VMEM_SHARED((acc_rows, BD), jnp.float32),
            pltpu.SemaphoreType.DMA,
            pltpu.SemaphoreType.DMA,
            pltpu.SemaphoreType.DMA,
            pltpu.SemaphoreType.DMA,
        ],
    )
    def matvec_sc(p_h, src_h, dst_h, rs_h, rd_h, out_h,
                  rsv0, rsv1, rdv0, rdv1, psv0, psv1, pdv0, pdv1,
                  csv0, csv1, cdv0, cdv1, sidx, didx, acc,
                  sem_in0, sem_in1, sem_out0, sem_out1):
        cid = lax.axis_index("c")
        sid = lax.axis_index("s")
        w = cid * NS + sid

        rsv = (rsv0, rsv1)
        rdv = (rdv0, rdv1)
        psv = (psv0, psv1)
        pdv = (pdv0, pdv1)
        csv = (csv0, csv1)
        cdv = (cdv0, cdv1)
        sem_in = (sem_in0, sem_in1)
        sem_out = (sem_out0, sem_out1)

        def _round_bf16(v):
            u = plsc.bitcast(v, jnp.uint32)
            u = (u + ((u >> 16) & jnp.uint32(1)) + jnp.uint32(0x7FFF))
            u = u & jnp.uint32(0xFFFF0000)
            return plsc.bitcast(u, jnp.float32)

        zvec = jnp.zeros((BD,), jnp.float32)

        @pl.loop(0, BLK)
        def _zero_buf(i):
            csv0[i] = zvec

        z0 = sid * acc_pt

        @pl.loop(0, acc_pt // BLK)
        def _zero_acc(k):
            pltpu.sync_copy(csv0, acc.at[pl.ds(z0 + k * BLK, BLK)])

        def issue_in(g, c, s):
            rbase = (w * chunks + g) * (A * D)
            pltpu.async_copy(rs_h.at[pl.ds(rbase, A * D)], rsv[s], sem_in[s])
            pltpu.async_copy(rd_h.at[pl.ds(rbase, A * D)], rdv[s], sem_in[s])
            pltpu.async_copy(p_h.at[sidx.at[c]], psv[s], sem_in[s])
            pltpu.async_copy(p_h.at[didx.at[c]], pdv[s], sem_in[s])

        def wait_in(s):
            pltpu.make_async_copy(rs_h.at[pl.ds(0, A * D)], rsv[s],
                                  sem_in[s]).wait()
            pltpu.make_async_copy(rd_h.at[pl.ds(0, A * D)], rdv[s],
                                  sem_in[s]).wait()
            pltpu.make_async_copy(p_h.at[pl.ds(0, BLK)], psv[s],
                                  sem_in[s]).wait()
            pltpu.make_async_copy(p_h.at[pl.ds(0, BLK)], pdv[s],
                                  sem_in[s]).wait()

        def issue_out(c, s):
            pltpu.async_copy(csv[s], acc.at[sidx.at[c]], sem_out[s],
                             add=True)
            pltpu.async_copy(cdv[s], acc.at[didx.at[c]], sem_out[s],
                             add=True)

        def wait_out(s):
            pltpu.make_async_copy(csv[s], acc.at[pl.ds(0, BLK)],
                                  sem_out[s]).wait()
            pltpu.make_async_copy(cdv[s], acc.at[pl.ds(0, BLK)],
                                  sem_out[s]).wait()

        e16 = jnp.arange(16, dtype=jnp.int32)

        def compute_chunk(s):
            @pl.loop(0, BLK // 16)
            def _blk(j):
                rows = j * 16 + e16

                def col(t):
                    return jnp.full((16,), t, jnp.int32)

                psT = [plsc.load_gather(psv[s], [rows, col(t)])
                       for t in range(BD)]
                pdT = [plsc.load_gather(pdv[s], [rows, col(t)])
                       for t in range(BD)]

                r = [[None] * A for _ in range(B)]
                for a in range(A):
                    for d in range(D):
                        k = a * D + d
                        rsk = rsv[s][k, pl.ds(j * 16, 16)]
                        rdk = rdv[s][k, pl.ds(j * 16, 16)]
                        for b in range(B):
                            t = rsk * psT[b * D + d] - rdk * pdT[b * D + d]
                            if r[b][a] is None:
                                r[b][a] = t
                            else:
                                r[b][a] = r[b][a] + t

                for b in range(B):
                    for a in range(A):
                        r[b][a] = _round_bf16(r[b][a])

                for d in range(D):
                    rs_cols = [rsv[s][a * D + d, pl.ds(j * 16, 16)]
                               for a in range(A)]
                    rd_cols = [rdv[s][a * D + d, pl.ds(j * 16, 16)]
                               for a in range(A)]
                    for b in range(B):
                        cs = rs_cols[0] * r[b][0]
                        cd = rd_cols[0] * r[b][0]
                        for a in range(1, A):
                            cs = cs + rs_cols[a] * r[b][a]
                            cd = cd + rd_cols[a] * r[b][a]
                        plsc.store_scatter(csv[s], [rows, col(b * D + d)], cs)
                        plsc.store_scatter(cdv[s], [rows, col(b * D + d)], -cd)

        plsc.subcore_barrier()

        def run_pass(g0):
            pltpu.sync_copy(src_h.at[pl.ds(w * chunks + g0, half)], sidx)
            pltpu.sync_copy(dst_h.at[pl.ds(w * chunks + g0, half)], didx)
            issue_in(g0, 0, 0)

            @pl.loop(0, half // 2)
            def _pair(i):
                c0 = i * 2
                wait_in(0)
                issue_in(g0 + c0 + 1, c0 + 1, 1)

                @pl.when(i > 0)
                def _():
                    wait_out(0)

                compute_chunk(0)
                issue_out(c0, 0)

                wait_in(1)

                @pl.when(c0 + 2 < half)
                def _():
                    issue_in(g0 + c0 + 2, c0 + 2, 0)

                @pl.when(i > 0)
                def _():
                    wait_out(1)

                compute_chunk(1)
                issue_out(c0 + 1, 1)

            wait_out(0)
            wait_out(1)

        run_pass(0)
        run_pass(half)

        plsc.subcore_barrier()
        pltpu.sync_copy(acc.at[pl.ds(z0, acc_pt)],
                        out_h.at[cid].at[pl.ds(z0, acc_pt)])

    return matvec_sc, acc_rows


def kernel(src, dst, R_src, R_dst, c0):
    e = src.shape[0]
    n = c0.shape[1]
    chunks = 4 * (-(-e // (W * BLK * 4)))
    e_pad = W * chunks * BLK

    matvec_sc, acc_rows = _build_matvec(n, e_pad, chunks)

    def bf(v):
        u = jax.lax.bitcast_convert_type(v, jnp.uint32)
        u = (u + ((u >> 16) & jnp.uint32(1)) + jnp.uint32(0x7FFF))
        u = u & jnp.uint32(0xFFFF0000)
        return jax.lax.bitcast_convert_type(u, jnp.float32)

    pad = e_pad - e
    src_p = jnp.pad(src.astype(jnp.int32), (0, pad)).reshape(e_pad // BLK, BLK)
    dst_p = jnp.pad(dst.astype(jnp.int32), (0, pad)).reshape(e_pad // BLK, BLK)
    nchunk = e_pad // BLK
    rs_p = (jnp.pad(bf(R_src).reshape(e, A * D), ((0, pad), (0, 0)))
            .reshape(nchunk, BLK, A * D).transpose(0, 2, 1)
            .reshape(nchunk * A * D, BLK))
    rd_p = (jnp.pad(bf(R_dst).reshape(e, A * D), ((0, pad), (0, 0)))
            .reshape(nchunk, BLK, A * D).transpose(0, 2, 1)
            .reshape(nchunk * A * D, BLK))

    def matvec(p):
        out = matvec_sc(bf(p), src_p, dst_p, rs_p, rd_p)
        return p + LAM * (out[0, :n] + out[1, :n])

    def bdot(u, v):
        s = (u * v).reshape(n, B, D).sum(axis=(0, 2))
        return jnp.repeat(s, D)[None, :]

    b = c0.astype(jnp.float32).transpose(1, 0, 2).reshape(n, BD)
    x = b
    r = b - matvec(x)
    p = r
    rsold = bdot(r, r)
    for _ in range(N_ITERS):
        ap = matvec(p)
        denom = bdot(p, ap) + 1e-12
        alpha = rsold / denom
        x = x + alpha * p
        r = r - alpha * ap
        rsnew = bdot(r, r)
        p = r + (rsnew / (rsold + 1e-12)) * p
        rsold = rsnew
    return x.reshape(n, B, D).transpose(1, 0, 2)
